# baseline (speedup 1.0000x reference)
import functools

import jax
import jax.numpy as jnp
from jax import lax
from jax.experimental import pallas as pl
from jax.experimental.pallas import tpu as pltpu
from jax.experimental.pallas import tpu_sc as plsc

B, P, V, D = 16384, 50, 39, 32
NC, NS = 2, 16
NW = NC * NS
BW = B // NW
KB = 16
NG = BW // KB
NB = 3


def _prep_body(x_ref, w_ref, pe_ref, c_ref, fused_ref):
    off = lax.broadcasted_iota(jnp.int32, (B, P), 1) * V
    c_ref[...] = x_ref[...] + off
    fused_ref[...] = pe_ref[0][:, None, :] + w_ref[...][None, :, :]


_prep = pl.pallas_call(
    _prep_body,
    out_shape=(
        jax.ShapeDtypeStruct((B, P), jnp.int32),
        jax.ShapeDtypeStruct((P, V, D), jnp.float32),
    ),
)


def _gather_body(fused_hbm, c_hbm, out_hbm, idx_v, rows_v, gsem, wsem):
    wid = lax.axis_index("s") * NC + lax.axis_index("c")
    pltpu.sync_copy(c_hbm.at[wid], idx_v)
    base = wid * BW

    def run_group(g, drain_prev_write):
        p = g % NB
        if drain_prev_write:
            pltpu.make_async_copy(
                rows_v.at[0], out_hbm.at[pl.ds(0, KB)], wsem
            ).wait()
        descs = [
            pltpu.async_copy(
                fused_hbm.at[idx_v.at[g * KB + k]],
                rows_v.at[p, k],
                gsem,
            )
            for k in range(KB)
        ]
        for d in descs:
            d.wait()
        pltpu.make_async_copy(
            rows_v.at[p], out_hbm.at[pl.ds(base + g * KB, KB)], wsem
        ).start()

    for g0 in range(NB):
        run_group(g0, False)

    def body(g, carry):
        run_group(g, True)
        return carry

    lax.fori_loop(NB, NG, body, 0)
    for _ in range(NB):
        pltpu.make_async_copy(
            rows_v.at[0], out_hbm.at[pl.ds(0, KB)], wsem
        ).wait()


_gather = functools.partial(
    pl.kernel,
    out_type=jax.ShapeDtypeStruct((B, P, D), jnp.float32),
    mesh=plsc.VectorSubcoreMesh(core_axis_name="c", subcore_axis_name="s"),
    scratch_types=[
        pltpu.VMEM((BW, P), jnp.int32),
        pltpu.VMEM((NB, KB, P, D), jnp.float32),
        pltpu.SemaphoreType.DMA,
        pltpu.SemaphoreType.DMA,
    ],
    compiler_params=pltpu.CompilerParams(use_tc_tiling_on_sc=False),
)(_gather_body)


def _probe_body(c_hbm, out_hbm, rows_v, wsem):
    wid = lax.axis_index("s") * NC + lax.axis_index("c")
    base = wid * BW

    def body(g, carry):
        pltpu.make_async_copy(
            rows_v.at[g % NB], out_hbm.at[pl.ds(base + g * KB, KB)], wsem
        ).start()
        return carry

    lax.fori_loop(0, NG, body, 0)

    def drain(g, carry):
        pltpu.make_async_copy(
            rows_v.at[0], out_hbm.at[pl.ds(0, KB)], wsem
        ).wait()
        return carry

    lax.fori_loop(0, NG, drain, 0)


_probe = functools.partial(
    pl.kernel,
    out_type=jax.ShapeDtypeStruct((B, D, P), jnp.float32),
    mesh=plsc.VectorSubcoreMesh(core_axis_name="c", subcore_axis_name="s"),
    scratch_types=[
        pltpu.VMEM((NB, KB, D, P), jnp.float32),
        pltpu.SemaphoreType.DMA,
    ],
    compiler_params=pltpu.CompilerParams(use_tc_tiling_on_sc=False),
)(_probe_body)


def kernel(x, embed_weight, pe):
    x = x.astype(jnp.int32)
    c, fused = _prep(x, embed_weight, pe)
    out = _gather(fused.reshape(P * V, D), c.reshape(NW, BW, P))
    return out
